# baseline (speedup 1.0000x reference)
import functools
import jax
import jax.numpy as jnp
from jax import lax
from jax.experimental import pallas as pl
from jax.experimental.pallas import tpu as pltpu
from jax.experimental.pallas import tpu_sc as plsc

_D, _F, _E = 1024, 2048, 8
_N = 2048
_TB = 512
_NB = _N * 2 // _TB + _E
_P = _NB * _TB
_AUX_W = 0.01
_NW = 32
_GCH = 64
_CCH = 32


def _router_body(x_ref, wr_ref, pos0_ref, pos1_ref, w0_ref, w1_ref,
                 bm_ref, aux_ref):
    xl = x_ref[...]
    iota = lax.broadcasted_iota(jnp.int32, (_N, 128), 1).astype(jnp.float32)
    logits = jnp.dot(xl, wr_ref[...], preferred_element_type=jnp.float32)
    logits = jnp.where(iota < _E, logits, -jnp.inf)
    m0 = jnp.max(logits, axis=1, keepdims=True)
    is0 = (logits == m0).astype(jnp.float32)
    i0 = 7.0 - jnp.max(is0 * (7.0 - iota) - (1.0 - is0) * 1e9, axis=1,
                       keepdims=True)
    oh0 = (iota == i0).astype(jnp.float32)
    masked = jnp.where(oh0 > 0, -jnp.inf, logits)
    m1 = jnp.max(masked, axis=1, keepdims=True)
    is1 = (masked == m1).astype(jnp.float32)
    i1 = 7.0 - jnp.max(is1 * (7.0 - iota) - (1.0 - is1) * 1e9, axis=1,
                       keepdims=True)
    oh1 = (iota == i1).astype(jnp.float32)
    w0 = 1.0 / (1.0 + jnp.exp(m1 - m0))
    w0_ref[...] = w0
    w1_ref[...] = 1.0 - w0

    p = jnp.exp(logits - m0)
    p = p / jnp.sum(p, axis=1, keepdims=True)
    avg_prob = jnp.mean(p, axis=0)
    avg_frac = jnp.mean(oh0, axis=0)
    aux = (_AUX_W * _E) * jnp.sum(avg_prob * avg_frac)
    aux_ref[...] = jnp.broadcast_to(aux, (1, 128))

    c0 = oh0
    c1 = oh1
    for s in (1, 2, 4, 8, 16, 32, 64, 128, 256, 512, 1024):
        c0 = c0 + jnp.concatenate(
            [jnp.zeros((s, 128), jnp.float32), c0[:-s]], axis=0)
        c1 = c1 + jnp.concatenate(
            [jnp.zeros((s, 128), jnp.float32), c1[:-s]], axis=0)
    ex0 = c0 - oh0
    ex1 = c1 - oh1
    cnt0 = jnp.sum(oh0, axis=0, keepdims=True)
    cnt1 = jnp.sum(oh1, axis=0, keepdims=True)
    counts = cnt0 + cnt1
    padded = jnp.floor((counts + (_TB - 1)) / _TB) * _TB
    ends = padded
    for s in (1, 2, 4, 8, 16, 32, 64):
        ends = ends + jnp.concatenate(
            [jnp.zeros((1, s), jnp.float32), ends[:, :-s]], axis=1)
    off = ends - padded
    pos0_ref[...] = jnp.sum(oh0 * (off + ex0), axis=1,
                            keepdims=True).astype(jnp.int32)
    pos1_ref[...] = jnp.sum(oh1 * (off + cnt0 + ex1), axis=1,
                            keepdims=True).astype(jnp.int32)

    r_iota = lax.broadcasted_iota(jnp.int32, (128, 128), 0).astype(jnp.float32)
    c_iota = lax.broadcasted_iota(jnp.int32, (128, 128), 1).astype(jnp.float32)
    covered = jnp.where(
        (jnp.broadcast_to(ends, (128, 128)) <= r_iota * _TB) & (c_iota < _E),
        1.0, 0.0)
    bm = jnp.minimum(jnp.sum(covered, axis=1, keepdims=True), 7.0)
    nvalid = jnp.sum(padded) / _TB
    r_col = lax.broadcasted_iota(jnp.int32, (128, 1), 0)
    bm_ref[...] = jnp.where(r_col == _NB, nvalid, bm).astype(jnp.int32)


@functools.cache
def _sc_mesh():
    return plsc.VectorSubcoreMesh(
        core_axis_name="c", subcore_axis_name="s",
        num_cores=2, num_subcores=16)


@functools.cache
def _make_dispatch():
    return functools.partial(
        pl.kernel,
        out_type=[
            jax.ShapeDtypeStruct((_P, _D), jnp.float32),
            jax.ShapeDtypeStruct((_P,), jnp.float32),
        ],
        mesh=_sc_mesh(),
        scratch_types=[
            pltpu.VMEM((_N,), jnp.int32),
            pltpu.VMEM((_N,), jnp.float32),
            pltpu.VMEM((_P,), jnp.int32),
            pltpu.VMEM((_P,), jnp.float32),
            pltpu.VMEM_SHARED((_P,), jnp.int32),
            pltpu.VMEM((_GCH,), jnp.int32),
            pltpu.VMEM((_GCH, _D), jnp.float32),
            pltpu.SemaphoreType.DMA,
        ],
        compiler_params=pltpu.CompilerParams(needs_layout_passes=False),
    )(_dispatch_body)


def _dispatch_body(pos0_hbm, pos1_hbm, w0_hbm, w1_hbm, x_hbm,
                   xs_hbm, ws_hbm,
                   pos_v, w_v, g_v, wsl_v, g_sh, idx_v, rows_v, sem):
    wid = lax.axis_index("s") * 2 + lax.axis_index("c")

    @pl.when(lax.axis_index("s") == 0)
    def _():
        def zero_body(i, carry):
            g_v[pl.ds(i * 16, 16)] = jnp.zeros((16,), jnp.int32)
            return carry
        lax.fori_loop(0, _P // 16, zero_body, 0)

        def scatter_pass(pos_hbm, w_hbm):
            pltpu.sync_copy(pos_hbm, pos_v)
            pltpu.sync_copy(w_hbm, w_v)

            def body(i, carry):
                idx = pos_v[pl.ds(i * 16, 16)]
                tok = lax.iota(jnp.int32, 16) + i * 16
                plsc.store_scatter(g_v, [idx], tok)
                plsc.store_scatter(wsl_v, [idx], w_v[pl.ds(i * 16, 16)])
                return carry
            lax.fori_loop(0, _N // 16, body, 0)

        scatter_pass(pos0_hbm, w0_hbm)
        scatter_pass(pos1_hbm, w1_hbm)
        pltpu.sync_copy(g_v, g_sh)

        @pl.when(lax.axis_index("c") == 0)
        def _():
            pltpu.sync_copy(wsl_v, ws_hbm)

    plsc.subcore_barrier()
    rows_per = _P // _NW
    base = wid * rows_per
    for c in range(rows_per // _GCH):
        lo = base + c * _GCH
        pltpu.sync_copy(g_sh.at[pl.ds(lo, _GCH)], idx_v)
        pltpu.async_copy(x_hbm.at[idx_v], rows_v, sem).wait()
        pltpu.sync_copy(rows_v, xs_hbm.at[pl.ds(lo, _GCH)])


def _ffn_body(bm_ref, x_ref, w1_ref, b1_ref, w2_ref, b2_ref, ws_ref, out_ref):
    i = pl.program_id(0)
    nvalid = bm_ref[_NB]

    @pl.when(i < nvalid)
    def _():
        h = jnp.dot(x_ref[...], w1_ref[0],
                    preferred_element_type=jnp.float32) + b1_ref[0]
        h = 0.5 * h * (1.0 + lax.erf(h * 0.7071067811865476))
        part = jnp.dot(h, w2_ref[0], preferred_element_type=jnp.float32)
        out_ref[...] = (part + b2_ref[0]) * ws_ref[...]


@functools.cache
def _make_combine():
    return functools.partial(
        pl.kernel,
        out_type=jax.ShapeDtypeStruct((_N, _D), jnp.float32),
        mesh=_sc_mesh(),
        scratch_types=[
            pltpu.VMEM((_CCH,), jnp.int32),
            pltpu.VMEM((_CCH,), jnp.int32),
            pltpu.VMEM((_CCH, _D), jnp.float32),
            pltpu.VMEM((_CCH, _D), jnp.float32),
            pltpu.SemaphoreType.DMA,
        ],
    )(_combine_body)


def _combine_body(pos0_hbm, pos1_hbm, y_hbm, out_hbm,
                  idx0_v, idx1_v, rows0_v, rows1_v, sem):
    wid = lax.axis_index("s") * 2 + lax.axis_index("c")
    tok_per = _N // _NW
    for c in range(tok_per // _CCH):
        lo = wid * tok_per + c * _CCH
        pltpu.sync_copy(pos0_hbm.at[pl.ds(lo, _CCH)], idx0_v)
        pltpu.sync_copy(pos1_hbm.at[pl.ds(lo, _CCH)], idx1_v)
        cp0 = pltpu.async_copy(y_hbm.at[idx0_v], rows0_v, sem)
        cp1 = pltpu.async_copy(y_hbm.at[idx1_v], rows1_v, sem)
        cp0.wait()
        cp1.wait()

        def add_row(r, carry):
            def add_vec(k, carry2):
                rows0_v[r, pl.ds(k * 16, 16)] = (
                    rows0_v[r, pl.ds(k * 16, 16)]
                    + rows1_v[r, pl.ds(k * 16, 16)])
                return carry2
            return lax.fori_loop(0, _D // 16, add_vec, carry)
        lax.fori_loop(0, _CCH, add_row, 0)
        pltpu.sync_copy(rows0_v, out_hbm.at[pl.ds(lo, _CCH)])


@jax.jit
def kernel(x, Wr, W1, b1, W2, b2):
    Bz, Tz, D = x.shape
    x_flat = x.reshape(-1, D)
    wr_pad = jnp.pad(Wr, ((0, 0), (0, 128 - _E)))

    pos0, pos1, w0, w1, bm, aux = pl.pallas_call(
        _router_body,
        out_shape=[
            jax.ShapeDtypeStruct((_N, 1), jnp.int32),
            jax.ShapeDtypeStruct((_N, 1), jnp.int32),
            jax.ShapeDtypeStruct((_N, 1), jnp.float32),
            jax.ShapeDtypeStruct((_N, 1), jnp.float32),
            jax.ShapeDtypeStruct((128, 1), jnp.int32),
            jax.ShapeDtypeStruct((1, 128), jnp.float32),
        ],
    )(x_flat, wr_pad)

    pos0f = pos0.reshape(_N)
    pos1f = pos1.reshape(_N)
    x_sorted, w_sorted = _make_dispatch()(pos0f, pos1f, w0.reshape(_N),
                                          w1.reshape(_N), x_flat)

    grid_spec = pltpu.PrefetchScalarGridSpec(
        num_scalar_prefetch=1,
        grid=(_NB,),
        in_specs=[
            pl.BlockSpec((_TB, _D), lambda i, bm: (i, 0)),
            pl.BlockSpec((1, _D, _F), lambda i, bm: (bm[i], 0, 0)),
            pl.BlockSpec((1, 1, _F), lambda i, bm: (bm[i], 0, 0)),
            pl.BlockSpec((1, _F, _D), lambda i, bm: (bm[i], 0, 0)),
            pl.BlockSpec((1, 1, _D), lambda i, bm: (bm[i], 0, 0)),
            pl.BlockSpec((_TB, 1), lambda i, bm: (i, 0)),
        ],
        out_specs=pl.BlockSpec((_TB, _D), lambda i, bm: (i, 0)),
    )
    y_sorted = pl.pallas_call(
        _ffn_body,
        grid_spec=grid_spec,
        out_shape=jax.ShapeDtypeStruct((_P, _D), jnp.float32),
        compiler_params=pltpu.CompilerParams(
            dimension_semantics=("arbitrary",),
        ),
    )(bm.reshape(128), x_sorted, W1, b1.reshape(_E, 1, _F), W2,
      b2.reshape(_E, 1, _D), w_sorted.reshape(_P, 1))

    out = _make_combine()(pos0f, pos1f, y_sorted)
    return out.reshape(Bz, Tz, D), aux[0, 0]
